# baseline (speedup 1.0000x reference)
import jax
import jax.numpy as jnp
from jax import lax
from jax.experimental import pallas as pl
from jax.experimental.pallas import tpu as pltpu
from jax.experimental.pallas import tpu_sc as plsc

N = 10000
E = 320000
D_IN = 128
D_OUT = 32
G = 64

NPAD = 10240
NC = 2
NS = 16
NW = NC * NS
EPW = E // NW
CHUNK = 128
NCHUNK = 80
EPW_PAD = NCHUNK * CHUNK
RPT = NPAD // NS
DEG_NBUF = 8
NBUF = 2

_mesh = plsc.VectorSubcoreMesh(core_axis_name="c", subcore_axis_name="s")


def _deg_body(ei_hbm, out_hbm, idx_v, ones_v, zeros_v, acc_sh, sem):
    c = lax.axis_index("c")
    s = lax.axis_index("s")
    wid = c * NS + s

    def _fill_zero(i, carry):
        zeros_v[pl.ds(i * 16, 16)] = jnp.zeros((16,), jnp.float32)
        return carry

    lax.fori_loop(0, RPT // 16, _fill_zero, 0)

    def _fill_one(i, carry):
        ones_v[pl.ds(i * 16, 16)] = jnp.ones((16,), jnp.float32)
        return carry

    lax.fori_loop(0, CHUNK // 16, _fill_one, 0)

    pltpu.sync_copy(zeros_v, acc_sh.at[pl.ds(s * RPT, RPT)])
    plsc.subcore_barrier()

    pltpu.sync_copy(ei_hbm.at[1, wid], idx_v)

    def _chunk(r, carry):
        for b in range(DEG_NBUF):
            pltpu.async_copy(ones_v, acc_sh.at[idx_v.at[r * DEG_NBUF + b]],
                             sem, add=True)
        for b in range(DEG_NBUF):
            pltpu.make_async_copy(ones_v,
                                  acc_sh.at[idx_v.at[r * DEG_NBUF + b]],
                                  sem).wait()
        return carry

    lax.fori_loop(0, NCHUNK // DEG_NBUF, _chunk, 0)
    plsc.subcore_barrier()
    pltpu.sync_copy(acc_sh.at[pl.ds(s * RPT, RPT)],
                    out_hbm.at[c, pl.ds(s * RPT, RPT)])


_deg_call = pl.kernel(
    _deg_body,
    out_type=jax.ShapeDtypeStruct((NC, NPAD), jnp.float32),
    mesh=_mesh,
    scratch_types=[
        pltpu.VMEM((NCHUNK, CHUNK), jnp.int32),
        pltpu.VMEM((CHUNK,), jnp.float32),
        pltpu.VMEM((RPT,), jnp.float32),
        pltpu.VMEM_SHARED((NPAD,), jnp.float32),
        pltpu.SemaphoreType.DMA,
    ],
)


_ZR = 64


def _scat_body(u_hbm, ei_hbm, out_hbm,
               idxs_b, idxd_b, rows_v, zrow_v, acc_sh,
               sem_i0, sem_i1, sem_g, sem_s):
    c = lax.axis_index("c")
    s = lax.axis_index("s")
    wid = c * NS + s

    @pl.when(c == 0)
    def _():
        pltpu.sync_copy(u_hbm.at[pl.ds(s * RPT, RPT)],
                        acc_sh.at[pl.ds(s * RPT, RPT)])

    @pl.when(c != 0)
    def _():
        def _fill_zero(r, carry):
            for k in range(D_IN // 16):
                zrow_v[r, pl.ds(16 * k, 16)] = jnp.zeros((16,), jnp.float32)
            return carry

        lax.fori_loop(0, _ZR, _fill_zero, 0)

        def _zero_acc(k, carry):
            pltpu.sync_copy(zrow_v, acc_sh.at[pl.ds(s * RPT + k * _ZR, _ZR)])
            return carry

        lax.fori_loop(0, RPT // _ZR, _zero_acc, 0)

    plsc.subcore_barrier()

    sems = (sem_i0, sem_i1)

    def idx_issue(slot, j):
        pltpu.async_copy(ei_hbm.at[0, wid, j], idxs_b.at[slot], sems[slot])
        pltpu.async_copy(ei_hbm.at[1, wid, j], idxd_b.at[slot], sems[slot])

    def idx_wait(slot):
        pltpu.make_async_copy(ei_hbm.at[0, wid, 0], idxs_b.at[slot],
                              sems[slot]).wait()
        pltpu.make_async_copy(ei_hbm.at[1, wid, 0], idxd_b.at[slot],
                              sems[slot]).wait()

    idx_issue(0, 0)
    idx_issue(1, 1)
    idx_wait(0)
    pltpu.async_copy(u_hbm.at[idxs_b.at[0]], rows_v.at[0], sem_g)

    def _round(sr, carry):
        for st in (0, 1):
            j = 2 * sr + st
            pltpu.make_async_copy(u_hbm.at[idxs_b.at[st]], rows_v.at[st],
                                  sem_g).wait()
            pltpu.async_copy(rows_v.at[st], acc_sh.at[idxd_b.at[st]],
                             sem_s, add=True)

            @pl.when(j + 1 < NCHUNK)
            def _():
                idx_wait(1 - st)
                pltpu.async_copy(u_hbm.at[idxs_b.at[1 - st]],
                                 rows_v.at[1 - st], sem_g)

            pltpu.make_async_copy(rows_v.at[st], acc_sh.at[idxd_b.at[st]],
                                  sem_s).wait()

            @pl.when(j + 2 < NCHUNK)
            def _():
                idx_issue(st, j + 2)
        return carry

    lax.fori_loop(0, NCHUNK // 2, _round, 0)
    plsc.subcore_barrier()
    pltpu.sync_copy(acc_sh.at[pl.ds(s * RPT, RPT)],
                    out_hbm.at[c, pl.ds(s * RPT, RPT)])


_scat_call = pl.kernel(
    _scat_body,
    out_type=jax.ShapeDtypeStruct((NC, NPAD, D_IN), jnp.float32),
    mesh=_mesh,
    scratch_types=[
        pltpu.VMEM((2, CHUNK), jnp.int32),
        pltpu.VMEM((2, CHUNK), jnp.int32),
        pltpu.VMEM((2, CHUNK, D_IN), jnp.float32),
        pltpu.VMEM((_ZR, D_IN), jnp.float32),
        pltpu.VMEM_SHARED((NPAD, D_IN), jnp.float32),
        pltpu.SemaphoreType.DMA,
        pltpu.SemaphoreType.DMA,
        pltpu.SemaphoreType.DMA,
        pltpu.SemaphoreType.DMA,
    ],
)


def _u_body(x_ref, d_ref, u_ref):
    deg = d_ref[0] + d_ref[1] + 1.0
    dinv = lax.rsqrt(deg)
    db = jnp.broadcast_to(dinv, (NPAD, D_IN))
    u_ref[0:N, :] = x_ref[...] * db[0:N, :]
    u_ref[N:NPAD, :] = jnp.zeros((NPAD - N, D_IN), jnp.float32)


def _u_call(x, degp3):
    return pl.pallas_call(
        _u_body,
        out_shape=jax.ShapeDtypeStruct((NPAD, D_IN), jnp.float32),
    )(x, degp3)


RBLK = 1000
NBLK = N // RBLK
GD = G * D_OUT


def _pool_body(batch_ref, s_ref, d_ref, wg_ref,
               bg_ref, w1_ref, wt2_ref, bc_ref, out_ref, macc, sacc, cacc):
    i = pl.program_id(0)

    @pl.when(i == 0)
    def _():
        macc[...] = jnp.full((1, GD), -jnp.inf, jnp.bfloat16)
        sacc[...] = jnp.zeros((G, D_OUT), jnp.float32)
        cacc[...] = jnp.zeros((G, D_OUT), jnp.float32)

    dinv = lax.rsqrt(d_ref[0] + d_ref[1] + 1.0)
    agg = ((s_ref[0] + s_ref[1]) *
           jnp.broadcast_to(dinv, (RBLK, D_IN)))
    y = lax.dot_general(agg.astype(jnp.bfloat16), wg_ref[...],
                        (((1,), (0,)), ((), ())),
                        preferred_element_type=jnp.float32)
    h = jnp.maximum(y + bg_ref[...], 0.0)

    onehot = (batch_ref[0] ==
              lax.broadcasted_iota(jnp.int32, (RBLK, G), 1)
              ).astype(jnp.float32)
    sacc[...] = sacc[...] + lax.dot_general(
        onehot, h, (((0,), (0,)), ((), ())),
        preferred_element_type=jnp.float32,
        precision=lax.Precision.HIGHEST)
    cacc[...] = cacc[...] + lax.dot_general(
        onehot, jnp.ones((RBLK, D_OUT), jnp.float32),
        (((0,), (0,)), ((), ())),
        preferred_element_type=jnp.float32,
        precision=lax.Precision.HIGHEST)

    hbf = h.astype(jnp.bfloat16)
    ht = jnp.broadcast_to(hbf[:, None, :], (RBLK, G, D_OUT)).reshape(RBLK, GD)
    gcol = lax.broadcasted_iota(jnp.int32, (1, GD), 1) // D_OUT
    mask = batch_ref[0] == gcol
    neg = jnp.array(-jnp.inf, jnp.bfloat16)
    macc[...] = jnp.maximum(
        macc[...],
        jnp.max(jnp.where(mask, ht, neg), axis=0, keepdims=True))

    @pl.when(i == NBLK - 1)
    def _():
        gap_bf = (sacc[...] / jnp.maximum(cacc[...], 1.0)).astype(jnp.bfloat16)
        out_gap = lax.dot_general(
            w1_ref[...].astype(jnp.bfloat16), gap_bf,
            (((1,), (1,)), ((), ())),
            preferred_element_type=jnp.float32)
        gmp_bf = jnp.maximum(macc[...], jnp.array(-1e30, jnp.bfloat16))
        contrib = (gmp_bf.astype(jnp.float32) *
                   wt2_ref[...].astype(jnp.bfloat16).astype(jnp.float32))
        iog = lax.broadcasted_iota(jnp.int32, (G, GD), 0)
        ioc = lax.broadcasted_iota(jnp.int32, (G, GD), 1) // D_OUT
        sel = (iog == ioc).astype(jnp.float32)
        out_max = lax.dot_general(contrib, sel,
                                  (((1,), (1,)), ((), ())),
                                  preferred_element_type=jnp.float32,
                                  precision=lax.Precision.HIGHEST)
        out_ref[...] = out_gap + out_max + bc_ref[...]


def _pool_call(batch3, s, degp3, wg, bg, w1, wt2, bc):
    full = lambda shape: pl.BlockSpec(shape, lambda i: tuple(0 for _ in shape))
    return pl.pallas_call(
        _pool_body,
        grid=(NBLK,),
        in_specs=[
            pl.BlockSpec((1, RBLK, 1), lambda i: (i, 0, 0)),
            pl.BlockSpec((NC, RBLK, D_IN), lambda i: (0, i, 0)),
            pl.BlockSpec((NC, RBLK, 1), lambda i: (0, i, 0)),
            full((D_IN, D_OUT)), full((1, D_OUT)), full((1, D_OUT)),
            full((1, GD)), full((1, G)),
        ],
        out_specs=full((1, G)),
        out_shape=jax.ShapeDtypeStruct((1, G), jnp.float32),
        scratch_shapes=[
            pltpu.VMEM((1, GD), jnp.bfloat16),
            pltpu.VMEM((G, D_OUT), jnp.float32),
            pltpu.VMEM((G, D_OUT), jnp.float32),
        ],
    )(batch3, s, degp3, wg, bg, w1, wt2, bc)


def kernel(x, edge_index, batch, W_gcn, b_gcn, W_cls, b_cls):
    pad_vals = N + (jnp.arange(EPW_PAD - EPW, dtype=jnp.int32) % (NPAD - N))
    pad = jnp.broadcast_to(pad_vals[None, None, :], (2, NW, EPW_PAD - EPW))
    ei = jnp.concatenate([edge_index.reshape(2, NW, EPW), pad],
                         axis=2).reshape(2, NW, NCHUNK, CHUNK)

    degp = _deg_call(ei)
    degp3 = degp.reshape(NC, NPAD, 1)
    u = _u_call(x, degp3)
    s = _scat_call(u, ei)

    batch3 = batch.reshape(NBLK, RBLK, 1)
    wgb = W_gcn.astype(jnp.bfloat16)
    w1 = W_cls[0:D_OUT, 0].reshape(1, D_OUT)
    wt2 = jnp.tile(W_cls[D_OUT:2 * D_OUT, 0], G).reshape(1, GD)
    out = _pool_call(batch3, s, degp3, wgb,
                     b_gcn.reshape(1, D_OUT), w1, wt2,
                     jnp.broadcast_to(b_cls.reshape(1, 1), (1, G)))
    return out.reshape(G, 1)


# speedup vs baseline: 1.0323x; 1.0323x over previous
import jax
import jax.numpy as jnp
from jax import lax
from jax.experimental import pallas as pl
from jax.experimental.pallas import tpu as pltpu
from jax.experimental.pallas import tpu_sc as plsc

N = 10000
E = 320000
D_IN = 128
D_OUT = 32
G = 64

NPAD = 10240
NC = 2
NS = 16
NW = NC * NS
EPW = E // NW
CHUNK = 128
NCHUNK = 80
EPW_PAD = NCHUNK * CHUNK
RPT = NPAD // NS
DEG_NBUF = 8
NBUF = 2

_mesh = plsc.VectorSubcoreMesh(core_axis_name="c", subcore_axis_name="s")


def _deg_body(ei_hbm, out_hbm, idx_v, ones_v, zeros_v, acc_sh, sem):
    c = lax.axis_index("c")
    s = lax.axis_index("s")
    wid = c * NS + s

    def _fill_zero(i, carry):
        zeros_v[pl.ds(i * 16, 16)] = jnp.zeros((16,), jnp.float32)
        return carry

    lax.fori_loop(0, RPT // 16, _fill_zero, 0)

    def _fill_one(i, carry):
        ones_v[pl.ds(i * 16, 16)] = jnp.ones((16,), jnp.float32)
        return carry

    lax.fori_loop(0, CHUNK // 16, _fill_one, 0)

    pltpu.sync_copy(zeros_v, acc_sh.at[pl.ds(s * RPT, RPT)])
    plsc.subcore_barrier()

    pltpu.sync_copy(ei_hbm.at[1, wid], idx_v)

    def _chunk(r, carry):
        for b in range(DEG_NBUF):
            pltpu.async_copy(ones_v, acc_sh.at[idx_v.at[r * DEG_NBUF + b]],
                             sem, add=True)
        for b in range(DEG_NBUF):
            pltpu.make_async_copy(ones_v,
                                  acc_sh.at[idx_v.at[r * DEG_NBUF + b]],
                                  sem).wait()
        return carry

    lax.fori_loop(0, NCHUNK // DEG_NBUF, _chunk, 0)
    plsc.subcore_barrier()
    pltpu.sync_copy(acc_sh.at[pl.ds(s * RPT, RPT)],
                    out_hbm.at[c, pl.ds(s * RPT, RPT)])


_deg_call = pl.kernel(
    _deg_body,
    out_type=jax.ShapeDtypeStruct((NC, NPAD), jnp.float32),
    mesh=_mesh,
    scratch_types=[
        pltpu.VMEM((NCHUNK, CHUNK), jnp.int32),
        pltpu.VMEM((CHUNK,), jnp.float32),
        pltpu.VMEM((RPT,), jnp.float32),
        pltpu.VMEM_SHARED((NPAD,), jnp.float32),
        pltpu.SemaphoreType.DMA,
    ],
)


_ZR = 64


def _scat_body(u_hbm, ei_hbm, out_hbm,
               idxs_b, idxd_b, rows_v, zrow_v, acc_sh,
               sem_i0, sem_i1, sem_g, sem_s):
    c = lax.axis_index("c")
    s = lax.axis_index("s")
    wid = c * NS + s

    @pl.when(c == 0)
    def _():
        pltpu.sync_copy(u_hbm.at[pl.ds(s * RPT, RPT)],
                        acc_sh.at[pl.ds(s * RPT, RPT)])

    @pl.when(c != 0)
    def _():
        def _fill_zero(r, carry):
            for k in range(D_IN // 16):
                zrow_v[r, pl.ds(16 * k, 16)] = jnp.zeros((16,), jnp.float32)
            return carry

        lax.fori_loop(0, _ZR, _fill_zero, 0)

        def _zero_acc(k, carry):
            pltpu.sync_copy(zrow_v, acc_sh.at[pl.ds(s * RPT + k * _ZR, _ZR)])
            return carry

        lax.fori_loop(0, RPT // _ZR, _zero_acc, 0)

    plsc.subcore_barrier()

    sems = (sem_i0, sem_i1)

    def idx_issue(slot, j):
        pltpu.async_copy(ei_hbm.at[0, wid, j], idxs_b.at[slot], sems[slot])
        pltpu.async_copy(ei_hbm.at[1, wid, j], idxd_b.at[slot], sems[slot])

    def idx_wait(slot):
        pltpu.make_async_copy(ei_hbm.at[0, wid, 0], idxs_b.at[slot],
                              sems[slot]).wait()
        pltpu.make_async_copy(ei_hbm.at[1, wid, 0], idxd_b.at[slot],
                              sems[slot]).wait()

    idx_issue(0, 0)
    idx_issue(1, 1)
    idx_wait(0)
    pltpu.async_copy(u_hbm.at[idxs_b.at[0]], rows_v.at[0], sem_g)

    def _round(sr, carry):
        for st in (0, 1):
            j = 2 * sr + st
            pltpu.make_async_copy(u_hbm.at[idxs_b.at[st]], rows_v.at[st],
                                  sem_g).wait()
            pltpu.async_copy(rows_v.at[st], acc_sh.at[idxd_b.at[st]],
                             sem_s, add=True)

            @pl.when(j + 1 < NCHUNK)
            def _():
                idx_wait(1 - st)
                pltpu.async_copy(u_hbm.at[idxs_b.at[1 - st]],
                                 rows_v.at[1 - st], sem_g)

            pltpu.make_async_copy(rows_v.at[st], acc_sh.at[idxd_b.at[st]],
                                  sem_s).wait()

            @pl.when(j + 2 < NCHUNK)
            def _():
                idx_issue(st, j + 2)
        return carry

    lax.fori_loop(0, NCHUNK // 2, _round, 0)
    plsc.subcore_barrier()
    pltpu.sync_copy(acc_sh.at[pl.ds(s * RPT, RPT)],
                    out_hbm.at[c, pl.ds(s * RPT, RPT)])


_scat_call = pl.kernel(
    _scat_body,
    out_type=jax.ShapeDtypeStruct((NC, NPAD, D_IN), jnp.float32),
    mesh=_mesh,
    scratch_types=[
        pltpu.VMEM((2, CHUNK), jnp.int32),
        pltpu.VMEM((2, CHUNK), jnp.int32),
        pltpu.VMEM((2, CHUNK, D_IN), jnp.float32),
        pltpu.VMEM((_ZR, D_IN), jnp.float32),
        pltpu.VMEM_SHARED((NPAD, D_IN), jnp.float32),
        pltpu.SemaphoreType.DMA,
        pltpu.SemaphoreType.DMA,
        pltpu.SemaphoreType.DMA,
        pltpu.SemaphoreType.DMA,
    ],
)


def _u_body(x_ref, d_ref, u_ref):
    deg = d_ref[0] + d_ref[1] + 1.0
    dinv = lax.rsqrt(deg)
    db = jnp.broadcast_to(dinv, (NPAD, D_IN))
    u_ref[0:N, :] = x_ref[...] * db[0:N, :]
    u_ref[N:NPAD, :] = jnp.zeros((NPAD - N, D_IN), jnp.float32)


def _u_call(x, degp3):
    return pl.pallas_call(
        _u_body,
        out_shape=jax.ShapeDtypeStruct((NPAD, D_IN), jnp.float32),
    )(x, degp3)


RBLK = 1000
NBLK = N // RBLK
GD = G * D_OUT


def _pool_body(batch_ref, s_ref, d_ref, wg_ref,
               bg_ref, w1_ref, wt2_ref, bc_ref, out_ref, macc, sacc, cacc):
    i = pl.program_id(0)

    @pl.when(i == 0)
    def _():
        macc[...] = jnp.full((1, GD), -jnp.inf, jnp.float32)
        sacc[...] = jnp.zeros((G, D_OUT), jnp.float32)
        cacc[...] = jnp.zeros((G, D_OUT), jnp.float32)

    dinv = lax.rsqrt(d_ref[0] + d_ref[1] + 1.0)
    agg = ((s_ref[0] + s_ref[1]) *
           jnp.broadcast_to(dinv, (RBLK, D_IN)))
    y = lax.dot_general(agg.astype(jnp.bfloat16), wg_ref[...],
                        (((1,), (0,)), ((), ())),
                        preferred_element_type=jnp.float32)
    h = jnp.maximum(y + bg_ref[...], 0.0)

    onehot = (batch_ref[0] ==
              lax.broadcasted_iota(jnp.int32, (RBLK, G), 1)
              ).astype(jnp.float32)
    sacc[...] = sacc[...] + lax.dot_general(
        onehot, h, (((0,), (0,)), ((), ())),
        preferred_element_type=jnp.float32,
        precision=lax.Precision.HIGHEST)
    cacc[...] = cacc[...] + lax.dot_general(
        onehot, jnp.ones((RBLK, D_OUT), jnp.float32),
        (((0,), (0,)), ((), ())),
        preferred_element_type=jnp.float32,
        precision=lax.Precision.HIGHEST)

    ht = jnp.broadcast_to(h[:, None, :], (RBLK, G, D_OUT)).reshape(RBLK, GD)
    gcol = lax.broadcasted_iota(jnp.int32, (1, GD), 1) // D_OUT
    mask = batch_ref[0] == gcol
    macc[...] = jnp.maximum(
        macc[...],
        jnp.max(jnp.where(mask, ht, -jnp.inf), axis=0, keepdims=True))

    @pl.when(i == NBLK - 1)
    def _():
        gap_bf = (sacc[...] / jnp.maximum(cacc[...], 1.0)).astype(jnp.bfloat16)
        out_gap = lax.dot_general(
            w1_ref[...].astype(jnp.bfloat16), gap_bf,
            (((1,), (1,)), ((), ())),
            preferred_element_type=jnp.float32)
        gmp_bf = jnp.maximum(macc[...], jnp.float32(-1e30)).astype(jnp.bfloat16)
        contrib = (gmp_bf.astype(jnp.float32) *
                   wt2_ref[...].astype(jnp.bfloat16).astype(jnp.float32))
        iog = lax.broadcasted_iota(jnp.int32, (G, GD), 0)
        ioc = lax.broadcasted_iota(jnp.int32, (G, GD), 1) // D_OUT
        sel = (iog == ioc).astype(jnp.float32)
        out_max = lax.dot_general(contrib, sel,
                                  (((1,), (1,)), ((), ())),
                                  preferred_element_type=jnp.float32,
                                  precision=lax.Precision.HIGHEST)
        out_ref[...] = out_gap + out_max + bc_ref[...]


def _pool_call(batch3, s, degp3, wg, bg, w1, wt2, bc):
    full = lambda shape: pl.BlockSpec(shape, lambda i: tuple(0 for _ in shape))
    return pl.pallas_call(
        _pool_body,
        grid=(NBLK,),
        in_specs=[
            pl.BlockSpec((1, RBLK, 1), lambda i: (i, 0, 0)),
            pl.BlockSpec((NC, RBLK, D_IN), lambda i: (0, i, 0)),
            pl.BlockSpec((NC, RBLK, 1), lambda i: (0, i, 0)),
            full((D_IN, D_OUT)), full((1, D_OUT)), full((1, D_OUT)),
            full((1, GD)), full((1, G)),
        ],
        out_specs=full((1, G)),
        out_shape=jax.ShapeDtypeStruct((1, G), jnp.float32),
        scratch_shapes=[
            pltpu.VMEM((1, GD), jnp.float32),
            pltpu.VMEM((G, D_OUT), jnp.float32),
            pltpu.VMEM((G, D_OUT), jnp.float32),
        ],
    )(batch3, s, degp3, wg, bg, w1, wt2, bc)


def kernel(x, edge_index, batch, W_gcn, b_gcn, W_cls, b_cls):
    pad_vals = N + (jnp.arange(EPW_PAD - EPW, dtype=jnp.int32) % (NPAD - N))
    pad = jnp.broadcast_to(pad_vals[None, None, :], (2, NW, EPW_PAD - EPW))
    ei = jnp.concatenate([edge_index.reshape(2, NW, EPW), pad],
                         axis=2).reshape(2, NW, NCHUNK, CHUNK)

    degp = _deg_call(ei)
    degp3 = degp.reshape(NC, NPAD, 1)
    u = _u_call(x, degp3)
    s = _scat_call(u, ei)

    batch3 = batch.reshape(NBLK, RBLK, 1)
    wgb = W_gcn.astype(jnp.bfloat16)
    w1 = W_cls[0:D_OUT, 0].reshape(1, D_OUT)
    wt2 = jnp.tile(W_cls[D_OUT:2 * D_OUT, 0], G).reshape(1, GD)
    out = _pool_call(batch3, s, degp3, wgb,
                     b_gcn.reshape(1, D_OUT), w1, wt2,
                     jnp.broadcast_to(b_cls.reshape(1, 1), (1, G)))
    return out.reshape(G, 1)
